# baseline (speedup 1.0000x reference)
import functools

import jax
import jax.numpy as jnp
from jax import lax
from jax.experimental import pallas as pl
from jax.experimental.pallas import tpu as pltpu
from jax.experimental.pallas import tpu_sc as plsc

BV = 400
CHUNK = 256
LANES = 16
GRP = CHUNK // LANES


@functools.lru_cache(maxsize=None)
def _build_t1(D, V):
    info = plsc.get_sparse_core_info()
    NC, NS = info.num_cores, info.num_subcores
    NW = NC * NS
    nblk = V // BV
    per_w = -(-nblk // NW)
    PITCH = BV + 1
    mesh = plsc.VectorSubcoreMesh(core_axis_name="c", subcore_axis_name="s")

    @functools.partial(
        pl.kernel,
        mesh=mesh,
        out_type=jax.ShapeDtypeStruct((V, D), jnp.float32),
        scratch_types=[
            [pltpu.VMEM((D, PITCH), jnp.float32) for _ in range(2)],
            [pltpu.VMEM((BV, D), jnp.float32) for _ in range(2)],
            [pltpu.SemaphoreType.DMA for _ in range(2)],
            [pltpu.SemaphoreType.DMA for _ in range(2)],
        ],
        compiler_params=pltpu.CompilerParams(
            use_tc_tiling_on_sc=False, needs_layout_passes=False),
    )
    def t1_k(wt_hbm, out_hbm, ibufs, obufs, isems, osems):
        wid = lax.axis_index("s") * NC + lax.axis_index("c")
        i16 = lax.iota(jnp.int32, LANES)

        def blk(m):
            return pl.multiple_of((wid + m * NW) * BV, 8)

        def load_cp(m, b):
            return pltpu.make_async_copy(
                wt_hbm.at[:, pl.ds(blk(m), BV)],
                ibufs[b].at[:, pl.ds(0, BV)], isems[b])

        def store_cp(m, b):
            return pltpu.make_async_copy(
                obufs[b], out_hbm.at[pl.ds(blk(m), BV)], osems[b])

        def in_range(m):
            return wid + m * NW < nblk

        def transpose(b):
            ibuf = ibufs[b]
            obuf = obufs[b]

            @plsc.parallel_loop(0, BV // LANES)
            def vgroup(g):
                for i in range(LANES):
                    v = g * LANES + i
                    vcol = jnp.full((LANES,), 0, jnp.int32) + v
                    for k in range(D // LANES):
                        vals = plsc.load_gather(ibuf, [i16 + k * LANES, vcol])
                        obuf[v, pl.ds(k * LANES, LANES)] = vals

        @pl.when(in_range(0))
        def _():
            load_cp(0, 0).start()

        @pl.when(in_range(1))
        def _():
            load_cp(1, 1).start()

        def step(t, carry):
            for b in range(2):
                m = 2 * t + b

                @pl.when(jnp.logical_and(m >= 2, in_range(m - 2)))
                def _():
                    store_cp(m - 2, b).wait()

                @pl.when(in_range(m))
                def _():
                    load_cp(m, b).wait()
                    transpose(b)
                    store_cp(m, b).start()

                    @pl.when(in_range(m + 2))
                    def _():
                        load_cp(m + 2, b).start()

            return carry

        lax.fori_loop(0, (per_w + 3) // 2, step, 0)

    return t1_k


@functools.lru_cache(maxsize=None)
def _build_gather(V, D, J, S):
    info = plsc.get_sparse_core_info()
    NC, NS = info.num_cores, info.num_subcores
    NW = NC * NS
    B = J * S
    cpj = S // CHUNK
    n_chunks = B // CHUNK
    assert n_chunks % (2 * NW) == 0
    cpw = n_chunks // NW
    assert cpj & (cpj - 1) == 0
    cpj_shift = cpj.bit_length() - 1
    PITCH = CHUNK + 1
    mesh = plsc.VectorSubcoreMesh(core_axis_name="c", subcore_axis_name="s")

    @functools.partial(
        pl.kernel,
        mesh=mesh,
        out_type=jax.ShapeDtypeStruct((J * D, S), jnp.float32),
        scratch_types=[
            [pltpu.VMEM((CHUNK,), jnp.int32) for _ in range(2)],
            [pltpu.VMEM((CHUNK, D), jnp.float32) for _ in range(2)],
            [pltpu.VMEM((D, PITCH), jnp.float32) for _ in range(2)],
            [pltpu.SemaphoreType.DMA for _ in range(2)],
            [pltpu.SemaphoreType.DMA for _ in range(2)],
        ],
        compiler_params=pltpu.CompilerParams(
            use_tc_tiling_on_sc=False, needs_layout_passes=False),
    )
    def gather_k(table_hbm, idx_hbm, out_hbm, idx_v,
                 gbufs, tbufs, gsems, ssems):
        wid = lax.axis_index("s") * NC + lax.axis_index("c")
        c0 = wid * cpw
        i16 = lax.iota(jnp.int32, LANES)

        def load_indices(c, b):
            pltpu.sync_copy(idx_hbm.at[pl.ds(c * CHUNK, CHUNK)], idx_v[b])

        def fire_gather(b):
            pltpu.make_async_copy(
                table_hbm.at[idx_v[b]], gbufs[b], gsems[b]).start()

        def wait_gather(b):
            pltpu.make_async_copy(
                table_hbm.at[idx_v[b]], gbufs[b], gsems[b]).wait()

        def store_cp(c, b):
            j = c >> cpj_shift
            s0 = (c & (cpj - 1)) * CHUNK
            return pltpu.make_async_copy(
                tbufs[b].at[:, pl.ds(0, CHUNK)],
                out_hbm.at[pl.ds(j * D, D), pl.ds(s0, CHUNK)], ssems[b])

        def transpose(b):
            gbuf = gbufs[b]
            tbuf = tbufs[b]

            @plsc.parallel_loop(0, GRP)
            def sgroup(g):
                for i in range(LANES):
                    s = g * LANES + i
                    scol = jnp.full((LANES,), 0, jnp.int32) + s
                    for k in range(D // LANES):
                        vals = plsc.load_gather(
                            gbuf.at[s], [i16 + k * LANES])
                        plsc.store_scatter(
                            tbuf, [i16 + k * LANES, scol], vals)

        load_indices(c0, 0)
        fire_gather(0)
        load_indices(c0 + 1, 1)
        fire_gather(1)

        def group(t, carry):
            for b in range(2):
                i = 2 * t + b
                c = c0 + i
                wait_gather(b)

                @pl.when(i >= 2)
                def _():
                    store_cp(c - 2, b).wait()

                transpose(b)
                store_cp(c, b).start()

                @pl.when(i + 2 < cpw)
                def _():
                    load_indices(c + 2, b)
                    fire_gather(b)

            return carry

        lax.fori_loop(0, cpw // 2, group, 0)
        store_cp(c0 + cpw - 2, 0).wait()
        store_cp(c0 + cpw - 1, 1).wait()

    return gather_k


def kernel(input, weight):
    B0, B1 = input.shape
    V, D = weight.shape
    idx = input.T.reshape(-1).astype(jnp.int32)
    table = _build_t1(D, V)(weight.T)
    out = _build_gather(V, D, B1, B0)(table, idx)
    return jnp.transpose(out.reshape(B1, D, B0), (2, 0, 1))


# speedup vs baseline: 5.3460x; 5.3460x over previous
import functools

import jax
import jax.numpy as jnp
from jax import lax
from jax.experimental import pallas as pl
from jax.experimental.pallas import tpu as pltpu
from jax.experimental.pallas import tpu_sc as plsc

CHUNK = 256
LANES = 16
GRP = CHUNK // LANES


@functools.lru_cache(maxsize=None)
def _build_gather(V, D, J, S):
    info = plsc.get_sparse_core_info()
    NC, NS = info.num_cores, info.num_subcores
    NW = NC * NS
    B = J * S
    cpj = S // CHUNK
    n_chunks = B // CHUNK
    assert n_chunks % (2 * NW) == 0
    cpw = n_chunks // NW
    assert cpj & (cpj - 1) == 0
    cpj_shift = cpj.bit_length() - 1
    PITCH = CHUNK + 1
    mesh = plsc.VectorSubcoreMesh(core_axis_name="c", subcore_axis_name="s")

    @functools.partial(
        pl.kernel,
        mesh=mesh,
        out_type=jax.ShapeDtypeStruct((J * D, S), jnp.float32),
        scratch_types=[
            [pltpu.VMEM((CHUNK,), jnp.int32) for _ in range(2)],
            [pltpu.VMEM((CHUNK, D), jnp.float32) for _ in range(2)],
            [pltpu.VMEM((D, PITCH), jnp.float32) for _ in range(2)],
            [pltpu.SemaphoreType.DMA for _ in range(2)],
            [pltpu.SemaphoreType.DMA for _ in range(2)],
        ],
        compiler_params=pltpu.CompilerParams(
            use_tc_tiling_on_sc=False, needs_layout_passes=False),
    )
    def gather_k(table_hbm, idx_hbm, out_hbm, idx_v,
                 gbufs, tbufs, gsems, ssems):
        wid = lax.axis_index("s") * NC + lax.axis_index("c")
        c0 = wid * cpw
        i16 = lax.iota(jnp.int32, LANES)

        def load_indices(c, b):
            pltpu.sync_copy(idx_hbm.at[pl.ds(c * CHUNK, CHUNK)], idx_v[b])

        def fire_gather(b):
            pltpu.make_async_copy(
                table_hbm.at[idx_v[b]], gbufs[b], gsems[b]).start()

        def wait_gather(b):
            pltpu.make_async_copy(
                table_hbm.at[idx_v[b]], gbufs[b], gsems[b]).wait()

        def store_cp(c, b):
            j = c >> cpj_shift
            s0 = (c & (cpj - 1)) * CHUNK
            return pltpu.make_async_copy(
                tbufs[b].at[:, pl.ds(0, CHUNK)],
                out_hbm.at[pl.ds(j * D, D), pl.ds(s0, CHUNK)], ssems[b])

        def transpose(b):
            gbuf = gbufs[b]
            tbuf = tbufs[b]

            @plsc.parallel_loop(0, GRP)
            def sgroup(g):
                for i in range(LANES):
                    s = g * LANES + i
                    scol = jnp.full((LANES,), 0, jnp.int32) + s
                    for k in range(D // LANES):
                        vals = plsc.load_gather(
                            gbuf.at[s], [i16 + k * LANES])
                        plsc.store_scatter(
                            tbuf, [i16 + k * LANES, scol], vals)

        load_indices(c0, 0)
        fire_gather(0)
        load_indices(c0 + 1, 1)
        fire_gather(1)

        def group(t, carry):
            for b in range(2):
                i = 2 * t + b
                c = c0 + i
                wait_gather(b)

                @pl.when(i >= 2)
                def _():
                    store_cp(c - 2, b).wait()

                transpose(b)
                store_cp(c, b).start()

                @pl.when(i + 2 < cpw)
                def _():
                    load_indices(c + 2, b)
                    fire_gather(b)

            return carry

        lax.fori_loop(0, cpw // 2, group, 0)
        store_cp(c0 + cpw - 2, 0).wait()
        store_cp(c0 + cpw - 1, 1).wait()

    return gather_k


def kernel(input, weight):
    B0, B1 = input.shape
    V, D = weight.shape
    idx = input.T.reshape(-1).astype(jnp.int32)
    out = _build_gather(V, D, B1, B0)(weight, idx)
    return jnp.transpose(out.reshape(B1, D, B0), (2, 0, 1))
